# baseline (speedup 1.0000x reference)
import functools

import jax
import jax.numpy as jnp
from jax import lax
from jax.experimental import pallas as pl
from jax.experimental.pallas import tpu as pltpu
from jax.experimental.pallas import tpu_sc as plsc

D_MODEL = 768
D_FF = 3072
E = 8
S = 2048
K = 2

BLK = 256
NSLOT = S * K
P = ((NSLOT + E * (BLK - 1) + BLK - 1) // BLK) * BLK
NB = P // BLK

NC = 2
NS = 16
NW = NC * NS

SPW = NSLOT // NW
DCH = 64
NDCH = SPW // DCH
TPW = S // NW
CCH = 16
NCCH = TPW // CCH

FCH = 768
NF1 = D_FF // FCH
MCH = 256
NF2 = D_MODEL // MCH
NF = NF1 + NF2
LCH = D_MODEL // 16


def _routing_metadata(topk_e, topk_w):
    e_flat = topk_e.reshape(-1).astype(jnp.int32)
    oh = e_flat[:, None] == jnp.arange(E, dtype=jnp.int32)[None, :]
    ranks = jnp.cumsum(oh.astype(jnp.int32), axis=0)
    counts = ranks[-1]
    padded = ((counts + BLK - 1) // BLK) * BLK
    ends = jnp.cumsum(padded).astype(jnp.int32)
    starts = ends - padded.astype(jnp.int32)
    rank_i = jnp.sum(jnp.where(oh, ranks, 0), axis=1) - 1
    pos = starts[e_flat] + rank_i
    total = ends[-1]
    blk_start = jnp.arange(NB, dtype=jnp.int32) * BLK
    be = jnp.sum((ends[None, :] <= blk_start[:, None]).astype(jnp.int32),
                 axis=1)
    valid = (blk_start < total).astype(jnp.int32)
    last_e = jnp.sum((ends <= total - 1).astype(jnp.int32))
    be = jnp.where(valid == 1, jnp.minimum(be, E - 1), last_e)
    return pos, be, valid


@functools.lru_cache(maxsize=None)
def _sc_kernels():
    mesh = plsc.VectorSubcoreMesh(
        core_axis_name="c", subcore_axis_name="s",
        num_cores=NC, num_subcores=NS)

    @functools.partial(
        pl.kernel,
        out_type=jax.ShapeDtypeStruct((P, D_MODEL), jnp.float32),
        mesh=mesh,
        scratch_types=[
            pltpu.VMEM((NDCH, DCH), jnp.int32),
            pltpu.VMEM((NDCH, DCH), jnp.int32),
            pltpu.VMEM((DCH, D_MODEL), jnp.float32),
            pltpu.VMEM((DCH, D_MODEL), jnp.float32),
            pltpu.SemaphoreType.DMA,
            pltpu.SemaphoreType.DMA,
            pltpu.SemaphoreType.DMA,
            pltpu.SemaphoreType.DMA,
        ],
    )
    def _sc_dispatch(x_hbm, tok_hbm, pos_hbm, out_hbm,
                     tok_v, pos_v, r0, r1, g0, g1, s0, s1):
        wid = lax.axis_index("s") * NC + lax.axis_index("c")
        pltpu.sync_copy(tok_hbm.at[wid], tok_v)
        pltpu.sync_copy(pos_hbm.at[wid], pos_v)
        bufs, gsem, ssem = (r0, r1), (g0, g1), (s0, s1)

        def gath(c):
            return pltpu.async_copy(
                x_hbm.at[tok_v.at[c]], bufs[c % 2], gsem[c % 2])

        def scat(c):
            return pltpu.async_copy(
                bufs[c % 2], out_hbm.at[pos_v.at[c]], ssem[c % 2])

        ga = gath(0)
        gb = gath(1)
        ga.wait()
        sa = scat(0)
        gb.wait()
        sb = scat(1)
        sa.wait()
        sb.wait()

    @functools.partial(
        pl.kernel,
        out_type=jax.ShapeDtypeStruct((S, D_MODEL), jnp.float32),
        mesh=mesh,
        scratch_types=[
            pltpu.VMEM((NCCH, K * CCH), jnp.int32),
            pltpu.VMEM((NCCH, K * CCH, 16), jnp.float32),
            pltpu.VMEM((K * CCH, D_MODEL), jnp.float32),
            pltpu.VMEM((K * CCH, D_MODEL), jnp.float32),
            pltpu.VMEM((CCH, D_MODEL), jnp.float32),
            pltpu.VMEM((CCH, D_MODEL), jnp.float32),
            pltpu.SemaphoreType.DMA,
            pltpu.SemaphoreType.DMA,
            pltpu.SemaphoreType.DMA,
            pltpu.SemaphoreType.DMA,
        ],
    )
    def _sc_combine(yg_hbm, pos_hbm, w_hbm, y_hbm,
                    pos_v, w_v, r0, r1, o0, o1, g0, g1, s0, s1):
        wid = lax.axis_index("s") * NC + lax.axis_index("c")
        tbase = wid * TPW
        pltpu.sync_copy(pos_hbm.at[wid], pos_v)
        pltpu.sync_copy(w_hbm.at[wid], w_v)
        rbufs, obufs, gsem, ssem = (r0, r1), (o0, o1), (g0, g1), (s0, s1)

        def gath(c):
            return pltpu.async_copy(
                yg_hbm.at[pos_v.at[c]], rbufs[c % 2], gsem[c % 2])

        def store(c):
            return pltpu.async_copy(
                obufs[c % 2], y_hbm.at[pl.ds(tbase + c * CCH, CCH)],
                ssem[c % 2])

        def weighted_pair_add(c):
            rv, ov = rbufs[c % 2], obufs[c % 2]

            def tok_body(t, _):
                w0 = w_v[c, 2 * t]
                w1_ = w_v[c, 2 * t + 1]
                for j in range(LCH):
                    a = rv[2 * t, pl.ds(j * 16, 16)]
                    b = rv[2 * t + 1, pl.ds(j * 16, 16)]
                    ov[t, pl.ds(j * 16, 16)] = a * w0 + b * w1_
                return 0

            lax.fori_loop(0, CCH, tok_body, 0)

        ga = gath(0)
        gb = gath(1)
        ga.wait()
        weighted_pair_add(0)
        sa = store(0)
        gc_ = gath(2)
        gb.wait()
        weighted_pair_add(1)
        sb = store(1)
        gd = gath(3)
        sa.wait()
        gc_.wait()
        weighted_pair_add(2)
        sc = store(2)
        sb.wait()
        gd.wait()
        weighted_pair_add(3)
        sd = store(3)
        sc.wait()
        sd.wait()

    return _sc_dispatch, _sc_combine


def _ffn_body(be_ref, vld_ref, xg_ref, w1_ref, w2_ref, out_ref, h_scr):
    b = pl.program_id(0)
    f = pl.program_id(1)

    @pl.when(jnp.logical_and(vld_ref[b] == 1, f < NF1))
    def _():
        x = xg_ref[...]
        h = lax.dot_general(
            x, w1_ref[0], (((1,), (1,)), ((), ())),
            preferred_element_type=jnp.float32)
        off = pl.multiple_of(f * FCH, FCH)
        h_scr[:, pl.ds(off, FCH)] = h * lax.logistic(h)

    @pl.when(jnp.logical_and(vld_ref[b] == 1, f >= NF1))
    def _():
        out_ref[...] = lax.dot_general(
            h_scr[...], w2_ref[0], (((1,), (1,)), ((), ())),
            preferred_element_type=jnp.float32)


def _grouped_ffn(xg, w1, w2, be, valid):
    grid_spec = pltpu.PrefetchScalarGridSpec(
        num_scalar_prefetch=2,
        grid=(NB, NF),
        in_specs=[
            pl.BlockSpec((BLK, D_MODEL), lambda b, f, be, vld: (b, 0)),
            pl.BlockSpec((1, FCH, D_MODEL),
                         lambda b, f, be, vld: (be[b], jnp.minimum(f, NF1 - 1),
                                                0)),
            pl.BlockSpec((1, MCH, D_FF),
                         lambda b, f, be, vld: (be[b],
                                                jnp.maximum(f - NF1, 0), 0)),
        ],
        out_specs=pl.BlockSpec(
            (BLK, MCH),
            lambda b, f, be, vld: (b, jnp.maximum(f - NF1, 0))),
        scratch_shapes=[pltpu.VMEM((BLK, D_FF), jnp.float32)],
    )
    return pl.pallas_call(
        _ffn_body,
        grid_spec=grid_spec,
        out_shape=jax.ShapeDtypeStruct((P, D_MODEL), jnp.float32),
        compiler_params=pltpu.CompilerParams(
            dimension_semantics=("arbitrary", "arbitrary"),
        ),
    )(be, valid, xg, w1, w2)


def kernel(x, topk_e, topk_w, w1, w2):
    sc_dispatch, sc_combine = _sc_kernels()
    pos, be, valid = _routing_metadata(topk_e, topk_w)
    tok_of_slot = (jnp.arange(NSLOT, dtype=jnp.int32) // K).reshape(
        NW, NDCH, DCH)
    xg = sc_dispatch(x, tok_of_slot, pos.reshape(NW, NDCH, DCH))
    yg = _grouped_ffn(xg, w1, w2, be, valid)
    w16 = jnp.broadcast_to(
        topk_w.reshape(-1).astype(jnp.float32)[:, None], (NSLOT, 16))
    y = sc_combine(yg, pos.reshape(NW, NCCH, K * CCH),
                   w16.reshape(NW, NCCH, K * CCH, 16))
    return y


# speedup vs baseline: 1.7162x; 1.7162x over previous
import functools

import jax
import jax.numpy as jnp
from jax import lax
from jax.experimental import pallas as pl
from jax.experimental.pallas import tpu as pltpu
from jax.experimental.pallas import tpu_sc as plsc

D_MODEL = 768
D_FF = 3072
E = 8
S = 2048
K = 2

BLK = 256
NSLOT = S * K
P = ((NSLOT + E * (BLK - 1) + BLK - 1) // BLK) * BLK
NB = P // BLK

NC = 2
NS = 16
NW = NC * NS

SPW = NSLOT // NW
DCH = 64
NDCH = SPW // DCH
TPW = S // NW
CCH = 32
NCCH = TPW // CCH
LCH = D_MODEL // 16
DM2 = D_MODEL // 2


def _routing_metadata(topk_e, topk_w):
    e_flat = topk_e.reshape(-1).astype(jnp.int32)
    oh = e_flat[:, None] == jnp.arange(E, dtype=jnp.int32)[None, :]
    ranks = jnp.cumsum(oh.astype(jnp.int32), axis=0)
    counts = ranks[-1]
    padded = ((counts + BLK - 1) // BLK) * BLK
    ends = jnp.cumsum(padded).astype(jnp.int32)
    starts = ends - padded.astype(jnp.int32)
    rank_i = jnp.sum(jnp.where(oh, ranks, 0), axis=1) - 1
    pos = starts[e_flat] + rank_i
    total = ends[-1]
    blk_start = jnp.arange(NB, dtype=jnp.int32) * BLK
    be = jnp.sum((ends[None, :] <= blk_start[:, None]).astype(jnp.int32),
                 axis=1)
    valid = (blk_start < total).astype(jnp.int32)
    last_e = jnp.sum((ends <= total - 1).astype(jnp.int32))
    be = jnp.where(valid == 1, jnp.minimum(be, E - 1), last_e)
    return pos, be, valid


@functools.lru_cache(maxsize=None)
def _sc_kernels():
    mesh = plsc.VectorSubcoreMesh(
        core_axis_name="c", subcore_axis_name="s",
        num_cores=NC, num_subcores=NS)

    @functools.partial(
        pl.kernel,
        out_type=jax.ShapeDtypeStruct((P, D_MODEL), jnp.float32),
        mesh=mesh,
        scratch_types=[
            pltpu.VMEM((NDCH, DCH), jnp.int32),
            pltpu.VMEM((NDCH, DCH), jnp.int32),
            pltpu.VMEM((DCH, D_MODEL), jnp.float32),
            pltpu.VMEM((DCH, D_MODEL), jnp.float32),
            pltpu.SemaphoreType.DMA,
            pltpu.SemaphoreType.DMA,
            pltpu.SemaphoreType.DMA,
            pltpu.SemaphoreType.DMA,
        ],
    )
    def _sc_dispatch(x_hbm, tok_hbm, pos_hbm, out_hbm,
                     tok_v, pos_v, r0, r1, g0, g1, s0, s1):
        wid = lax.axis_index("s") * NC + lax.axis_index("c")
        pltpu.sync_copy(tok_hbm.at[wid], tok_v)
        pltpu.sync_copy(pos_hbm.at[wid], pos_v)
        bufs, gsem, ssem = (r0, r1), (g0, g1), (s0, s1)

        def gath(c):
            return pltpu.async_copy(
                x_hbm.at[tok_v.at[c]], bufs[c % 2], gsem[c % 2])

        def scat(c):
            return pltpu.async_copy(
                bufs[c % 2], out_hbm.at[pos_v.at[c]], ssem[c % 2])

        ga = gath(0)
        gb = gath(1)
        ga.wait()
        sa = scat(0)
        gb.wait()
        sb = scat(1)
        sa.wait()
        sb.wait()

    @functools.partial(
        pl.kernel,
        out_type=jax.ShapeDtypeStruct((S, DM2), jnp.int32),
        mesh=mesh,
        compiler_params=pltpu.CompilerParams(needs_layout_passes=False),
        scratch_types=[
            pltpu.VMEM((NCCH, K * CCH), jnp.int32),
            pltpu.VMEM((NCCH, K * CCH, 16), jnp.float32),
            pltpu.VMEM((K * CCH, DM2), jnp.int32),
            pltpu.VMEM((K * CCH, DM2), jnp.int32),
            pltpu.VMEM((CCH, DM2), jnp.int32),
            pltpu.VMEM((CCH, DM2), jnp.int32),
            pltpu.SemaphoreType.DMA,
            pltpu.SemaphoreType.DMA,
            pltpu.SemaphoreType.DMA,
            pltpu.SemaphoreType.DMA,
        ],
    )
    def _sc_combine(yg_hbm, pos_hbm, w_hbm, y_hbm,
                    pos_v, w_v, r0, r1, o0, o1, g0, g1, s0, s1):
        wid = lax.axis_index("s") * NC + lax.axis_index("c")
        tbase = wid * TPW
        pltpu.sync_copy(pos_hbm.at[wid], pos_v)
        pltpu.sync_copy(w_hbm.at[wid], w_v)
        rbufs, obufs, gsem, ssem = (r0, r1), (o0, o1), (g0, g1), (s0, s1)

        def gath(c):
            return pltpu.async_copy(
                yg_hbm.at[pos_v.at[c]], rbufs[c % 2], gsem[c % 2])

        def store(c):
            return pltpu.async_copy(
                obufs[c % 2], y_hbm.at[pl.ds(tbase + c * CCH, CCH)],
                ssem[c % 2])

        hi_mask = jnp.full((16,), jnp.int32(-65536))
        lo_mask = jnp.full((16,), jnp.int32(0xFFFF))
        rnd = jnp.full((16,), jnp.int32(0x8000))

        def halves(word):
            lo = plsc.bitcast(word << 16, jnp.float32)
            hi = plsc.bitcast(word & hi_mask, jnp.float32)
            return lo, hi

        def weighted_pair_add(c):
            rv, ov = rbufs[c % 2], obufs[c % 2]

            def tok_body(t, _):
                w0 = w_v[c, 2 * t]
                w1_ = w_v[c, 2 * t + 1]
                for j in range(DM2 // 16):
                    a = rv[2 * t, pl.ds(j * 16, 16)]
                    b = rv[2 * t + 1, pl.ds(j * 16, 16)]
                    alo, ahi = halves(a)
                    blo, bhi = halves(b)
                    rlo = plsc.bitcast(alo * w0 + blo * w1_, jnp.int32)
                    rhi = plsc.bitcast(ahi * w0 + bhi * w1_, jnp.int32)
                    packed = ((rhi + rnd) & hi_mask) | (
                        lax.shift_right_logical(rlo + rnd, 16) & lo_mask)
                    ov[t, pl.ds(j * 16, 16)] = packed
                return 0

            lax.fori_loop(0, CCH, tok_body, 0)

        ga = gath(0)
        gb = gath(1)
        ga.wait()
        weighted_pair_add(0)
        sa = store(0)
        gb.wait()
        sa.wait()
        weighted_pair_add(1)
        sb = store(1)
        sb.wait()

    return _sc_dispatch, _sc_combine


def _ffn_body(be_ref, vld_ref, xg_ref, w1_ref, w2_ref, out_ref):
    b = pl.program_id(0)

    @pl.when(vld_ref[b] == 1)
    def _():
        x = xg_ref[...]
        h = lax.dot_general(
            x, w1_ref[0], (((1,), (1,)), ((), ())),
            preferred_element_type=jnp.float32)
        h = h * lax.logistic(h)
        yp = lax.dot_general(
            h, w2_ref[0], (((1,), (1,)), ((), ())),
            preferred_element_type=jnp.float32)
        yi = lax.bitcast_convert_type(yp, jnp.int32)
        lo = lax.shift_right_logical(yi[:, :DM2] + 0x8000, 16) & 0xFFFF
        hi = (yi[:, DM2:] + 0x8000) & jnp.int32(-65536)
        out_ref[...] = hi | lo


def _grouped_ffn(xg, w1, w2, be, valid):
    grid_spec = pltpu.PrefetchScalarGridSpec(
        num_scalar_prefetch=2,
        grid=(NB,),
        in_specs=[
            pl.BlockSpec((BLK, D_MODEL), lambda b, be, vld: (b, 0)),
            pl.BlockSpec((1, D_FF, D_MODEL), lambda b, be, vld: (be[b], 0, 0)),
            pl.BlockSpec((1, D_MODEL, D_FF), lambda b, be, vld: (be[b], 0, 0)),
        ],
        out_specs=pl.BlockSpec((BLK, DM2), lambda b, be, vld: (b, 0)),
    )
    return pl.pallas_call(
        _ffn_body,
        grid_spec=grid_spec,
        out_shape=jax.ShapeDtypeStruct((P, DM2), jnp.int32),
        compiler_params=pltpu.CompilerParams(
            dimension_semantics=("arbitrary",),
        ),
    )(be, valid, xg, w1, w2)


def kernel(x, topk_e, topk_w, w1, w2):
    sc_dispatch, sc_combine = _sc_kernels()
    pos, be, valid = _routing_metadata(topk_e, topk_w)
    tok_of_slot = (jnp.arange(NSLOT, dtype=jnp.int32) // K).reshape(
        NW, NDCH, DCH)
    xg = sc_dispatch(x, tok_of_slot, pos.reshape(NW, NDCH, DCH))
    yg = _grouped_ffn(xg, w1, w2, be, valid)
    w16 = jnp.broadcast_to(
        topk_w.reshape(-1).astype(jnp.float32)[:, None], (NSLOT, 16))
    yp = sc_combine(yg, pos.reshape(NW, NCCH, K * CCH),
                    w16.reshape(NW, NCCH, K * CCH, 16))
    y_lo = lax.bitcast_convert_type(yp << 16, jnp.float32)
    y_hi = lax.bitcast_convert_type(yp & jnp.int32(-65536), jnp.float32)
    return jnp.concatenate([y_lo, y_hi], axis=1)


# speedup vs baseline: 1.7874x; 1.0414x over previous
import functools

import jax
import jax.numpy as jnp
from jax import lax
from jax.experimental import pallas as pl
from jax.experimental.pallas import tpu as pltpu
from jax.experimental.pallas import tpu_sc as plsc

D_MODEL = 768
D_FF = 3072
E = 8
S = 2048
K = 2

BLK = 256
NSLOT = S * K
P = ((NSLOT + E * (BLK - 1) + BLK - 1) // BLK) * BLK
NB = P // BLK

NC = 2
NS = 16
NW = NC * NS

SPW = NSLOT // NW
DCH = 64
NDCH = SPW // DCH
TPW = S // NW
CCH = 32
NCCH = TPW // CCH
LCH = D_MODEL // 16
DM2 = D_MODEL // 2


def _routing_metadata(topk_e, topk_w):
    e_flat = topk_e.reshape(-1).astype(jnp.int32)
    oh = e_flat[:, None] == jnp.arange(E, dtype=jnp.int32)[None, :]
    ranks = jnp.cumsum(oh.astype(jnp.int32), axis=0)
    counts = ranks[-1]
    padded = ((counts + BLK - 1) // BLK) * BLK
    ends = jnp.cumsum(padded).astype(jnp.int32)
    starts = ends - padded.astype(jnp.int32)
    rank_i = jnp.sum(jnp.where(oh, ranks, 0), axis=1) - 1
    pos = starts[e_flat] + rank_i
    total = ends[-1]
    blk_start = jnp.arange(NB, dtype=jnp.int32) * BLK
    be = jnp.sum((ends[None, :] <= blk_start[:, None]).astype(jnp.int32),
                 axis=1)
    valid = (blk_start < total).astype(jnp.int32)
    last_e = jnp.sum((ends <= total - 1).astype(jnp.int32))
    be = jnp.where(valid == 1, jnp.minimum(be, E - 1), last_e)
    return pos, be, valid


@functools.lru_cache(maxsize=None)
def _sc_kernels():
    mesh = plsc.VectorSubcoreMesh(
        core_axis_name="c", subcore_axis_name="s",
        num_cores=NC, num_subcores=NS)

    @functools.partial(
        pl.kernel,
        out_type=jax.ShapeDtypeStruct((P, D_MODEL), jnp.float32),
        mesh=mesh,
        scratch_types=[
            pltpu.VMEM((NDCH, DCH), jnp.int32),
            pltpu.VMEM((NDCH, DCH), jnp.int32),
            pltpu.VMEM((DCH, D_MODEL), jnp.float32),
            pltpu.VMEM((DCH, D_MODEL), jnp.float32),
            pltpu.SemaphoreType.DMA,
            pltpu.SemaphoreType.DMA,
            pltpu.SemaphoreType.DMA,
            pltpu.SemaphoreType.DMA,
        ],
    )
    def _sc_dispatch(x_hbm, tok_hbm, pos_hbm, out_hbm,
                     tok_v, pos_v, r0, r1, g0, g1, s0, s1):
        wid = lax.axis_index("s") * NC + lax.axis_index("c")
        pltpu.sync_copy(tok_hbm.at[wid], tok_v)
        pltpu.sync_copy(pos_hbm.at[wid], pos_v)
        bufs, gsem, ssem = (r0, r1), (g0, g1), (s0, s1)

        def gath(c):
            return pltpu.async_copy(
                x_hbm.at[tok_v.at[c]], bufs[c % 2], gsem[c % 2])

        def scat(c):
            return pltpu.async_copy(
                bufs[c % 2], out_hbm.at[pos_v.at[c]], ssem[c % 2])

        ga = gath(0)
        gb = gath(1)
        ga.wait()
        sa = scat(0)
        gb.wait()
        sb = scat(1)
        sa.wait()
        sb.wait()

    @functools.partial(
        pl.kernel,
        out_type=jax.ShapeDtypeStruct((S, D_MODEL), jnp.float32),
        mesh=mesh,
        compiler_params=pltpu.CompilerParams(needs_layout_passes=False),
        scratch_types=[
            pltpu.VMEM((NCCH, K * CCH), jnp.int32),
            pltpu.VMEM((NCCH, K * CCH, 16), jnp.float32),
            pltpu.VMEM((K * CCH, DM2), jnp.int32),
            pltpu.VMEM((K * CCH, DM2), jnp.int32),
            pltpu.VMEM((CCH, D_MODEL), jnp.float32),
            pltpu.VMEM((CCH, D_MODEL), jnp.float32),
            pltpu.SemaphoreType.DMA,
            pltpu.SemaphoreType.DMA,
            pltpu.SemaphoreType.DMA,
            pltpu.SemaphoreType.DMA,
        ],
    )
    def _sc_combine(yg_hbm, pos_hbm, w_hbm, y_hbm,
                    pos_v, w_v, r0, r1, o0, o1, g0, g1, s0, s1):
        wid = lax.axis_index("s") * NC + lax.axis_index("c")
        tbase = wid * TPW
        pltpu.sync_copy(pos_hbm.at[wid], pos_v)
        pltpu.sync_copy(w_hbm.at[wid], w_v)
        rbufs, obufs, gsem, ssem = (r0, r1), (o0, o1), (g0, g1), (s0, s1)

        def gath(c):
            return pltpu.async_copy(
                yg_hbm.at[pos_v.at[c]], rbufs[c % 2], gsem[c % 2])

        def store(c):
            return pltpu.async_copy(
                obufs[c % 2], y_hbm.at[pl.ds(tbase + c * CCH, CCH)],
                ssem[c % 2])

        hi_mask = jnp.full((16,), jnp.int32(-65536))

        def halves(word):
            lo = plsc.bitcast(word << 16, jnp.float32)
            hi = plsc.bitcast(word & hi_mask, jnp.float32)
            return lo, hi

        def weighted_pair_add(c):
            rv, ov = rbufs[c % 2], obufs[c % 2]

            def tok_body(t, _):
                w0 = w_v[c, 2 * t]
                w1_ = w_v[c, 2 * t + 1]
                for j in range(DM2 // 16):
                    a = rv[2 * t, pl.ds(j * 16, 16)]
                    b = rv[2 * t + 1, pl.ds(j * 16, 16)]
                    alo, ahi = halves(a)
                    blo, bhi = halves(b)
                    ov[t, pl.ds(j * 16, 16)] = alo * w0 + blo * w1_
                    ov[t, pl.ds(DM2 + j * 16, 16)] = ahi * w0 + bhi * w1_
                return 0

            lax.fori_loop(0, CCH, tok_body, 0)

        ga = gath(0)
        gb = gath(1)
        ga.wait()
        weighted_pair_add(0)
        sa = store(0)
        gb.wait()
        sa.wait()
        weighted_pair_add(1)
        sb = store(1)
        sb.wait()

    return _sc_dispatch, _sc_combine


def _ffn_body(be_ref, vld_ref, xg_ref, w1_ref, w2_ref, out_ref):
    b = pl.program_id(0)

    @pl.when(vld_ref[b] == 1)
    def _():
        x = xg_ref[...]
        h = lax.dot_general(
            x, w1_ref[0], (((1,), (1,)), ((), ())),
            preferred_element_type=jnp.float32)
        h = h * lax.logistic(h)
        yp = lax.dot_general(
            h, w2_ref[0], (((1,), (1,)), ((), ())),
            preferred_element_type=jnp.float32)
        yi = lax.bitcast_convert_type(yp, jnp.int32)
        lo = lax.shift_right_logical(yi[:, :DM2] + 0x8000, 16) & 0xFFFF
        hi = (yi[:, DM2:] + 0x8000) & jnp.int32(-65536)
        out_ref[...] = hi | lo


def _grouped_ffn(xg, w1, w2, be, valid):
    grid_spec = pltpu.PrefetchScalarGridSpec(
        num_scalar_prefetch=2,
        grid=(NB,),
        in_specs=[
            pl.BlockSpec((BLK, D_MODEL), lambda b, be, vld: (b, 0)),
            pl.BlockSpec((1, D_FF, D_MODEL), lambda b, be, vld: (be[b], 0, 0)),
            pl.BlockSpec((1, D_MODEL, D_FF), lambda b, be, vld: (be[b], 0, 0)),
        ],
        out_specs=pl.BlockSpec((BLK, DM2), lambda b, be, vld: (b, 0)),
    )
    return pl.pallas_call(
        _ffn_body,
        grid_spec=grid_spec,
        out_shape=jax.ShapeDtypeStruct((P, DM2), jnp.int32),
        compiler_params=pltpu.CompilerParams(
            dimension_semantics=("arbitrary",),
        ),
    )(be, valid, xg, w1, w2)


def kernel(x, topk_e, topk_w, w1, w2):
    sc_dispatch, sc_combine = _sc_kernels()
    pos, be, valid = _routing_metadata(topk_e, topk_w)
    tok_of_slot = (jnp.arange(NSLOT, dtype=jnp.int32) // K).reshape(
        NW, NDCH, DCH)
    xg = sc_dispatch(x, tok_of_slot, pos.reshape(NW, NDCH, DCH))
    yg = _grouped_ffn(xg, w1, w2, be, valid)
    w16 = jnp.broadcast_to(
        topk_w.reshape(-1).astype(jnp.float32)[:, None], (NSLOT, 16))
    return sc_combine(yg, pos.reshape(NW, NCCH, K * CCH),
                      w16.reshape(NW, NCCH, K * CCH, 16))
